# Initial kernel scaffold; baseline (speedup 1.0000x reference)
import jax
import jax.numpy as jnp
from jax.experimental import pallas as pl


def kernel(x, edge_index, W1, b1, W2, b2):
    raise NotImplementedError("write your pallas kernel here")


# speedup vs baseline: 11.2859x; 11.2859x over previous
import functools
from functools import partial

import jax
import jax.numpy as jnp
from jax import lax
from jax.experimental import pallas as pl
from jax.experimental.pallas import tpu as pltpu
from jax.experimental.pallas import tpu_sc as plsc

NC = 2
NS = 16
NW = NC * NS
CHUNK = 128

_HIGH = jax.lax.Precision.HIGHEST


def _mesh():
    return plsc.VectorSubcoreMesh(
        core_axis_name="c", subcore_axis_name="s", num_cores=NC, num_subcores=NS
    )


def _hist_body(cpw, rpt, dst_hbm, zeros_hbm, ones_hbm, degp_hbm,
               idx_v, buf_v, acc_sh):
    c = lax.axis_index("c")
    s = lax.axis_index("s")
    wid = s * NC + c
    pltpu.sync_copy(zeros_hbm, acc_sh.at[pl.ds(s * rpt, rpt)])
    pltpu.sync_copy(ones_hbm, buf_v)
    pltpu.sync_copy(dst_hbm.at[pl.ds(wid * cpw, cpw)], idx_v)
    plsc.subcore_barrier()

    def body(j, carry):
        pltpu.sync_copy(buf_v, acc_sh.at[idx_v.at[j]], add=True)
        return carry

    lax.fori_loop(0, cpw, body, 0)
    plsc.subcore_barrier()
    pltpu.sync_copy(acc_sh.at[pl.ds(s * rpt, rpt)],
                    degp_hbm.at[c, pl.ds(s * rpt, rpt)])


def _make_hist(cpw, rows, rpt):
    return pl.kernel(
        partial(_hist_body, cpw, rpt),
        out_type=jax.ShapeDtypeStruct((NC, rows, 16), jnp.float32),
        mesh=_mesh(),
        compiler_params=pltpu.CompilerParams(use_tc_tiling_on_sc=False),
        scratch_types=[
            pltpu.VMEM((cpw, CHUNK), jnp.int32),
            pltpu.VMEM((CHUNK, 16), jnp.float32),
            pltpu.VMEM_SHARED((rows, 16), jnp.float32),
        ],
    )


def _agg_body(cpw, rpt, d, src_hbm, dst_hbm, table_hbm, zeros_hbm, part_hbm,
              sidx_v, didx_v, rbuf_v, acc_sh):
    c = lax.axis_index("c")
    s = lax.axis_index("s")
    wid = s * NC + c
    pltpu.sync_copy(zeros_hbm, acc_sh.at[pl.ds(s * rpt, rpt)])
    pltpu.sync_copy(src_hbm.at[pl.ds(wid * cpw, cpw)], sidx_v)
    pltpu.sync_copy(dst_hbm.at[pl.ds(wid * cpw, cpw)], didx_v)
    plsc.subcore_barrier()

    def body(j, carry):
        pltpu.sync_copy(table_hbm.at[sidx_v.at[j]], rbuf_v)
        pltpu.sync_copy(rbuf_v, acc_sh.at[didx_v.at[j]], add=True)
        return carry

    lax.fori_loop(0, cpw, body, 0)
    plsc.subcore_barrier()
    pltpu.sync_copy(acc_sh.at[pl.ds(s * rpt, rpt)],
                    part_hbm.at[c, pl.ds(s * rpt, rpt)])


def _make_agg(cpw, rows, rpt, d):
    return pl.kernel(
        partial(_agg_body, cpw, rpt, d),
        out_type=jax.ShapeDtypeStruct((NC, rows, d), jnp.float32),
        mesh=_mesh(),
        compiler_params=pltpu.CompilerParams(use_tc_tiling_on_sc=False),
        scratch_types=[
            pltpu.VMEM((cpw, CHUNK), jnp.int32),
            pltpu.VMEM((cpw, CHUNK), jnp.int32),
            pltpu.VMEM((CHUNK, d), jnp.float32),
            pltpu.VMEM_SHARED((rows, d), jnp.float32),
        ],
    )


def _scale_body(degp_ref, x_ref, dinv_ref, y1_ref):
    d = 1.0 + degp_ref[0, :, 0:1] + degp_ref[1, :, 0:1]
    dv = jax.lax.rsqrt(d)
    dinv_ref[...] = dv
    y1_ref[...] = dv * x_ref[...]


def _make_scale(n, rows, in_c, blk):
    grid = -(-rows // blk)
    return pl.pallas_call(
        _scale_body,
        grid=(grid,),
        in_specs=[
            pl.BlockSpec((NC, blk, 16), lambda i: (0, i, 0)),
            pl.BlockSpec((blk, in_c), lambda i: (i, 0)),
        ],
        out_specs=[
            pl.BlockSpec((blk, 1), lambda i: (i, 0)),
            pl.BlockSpec((blk, in_c), lambda i: (i, 0)),
        ],
        out_shape=[
            jax.ShapeDtypeStruct((rows, 1), jnp.float32),
            jax.ShapeDtypeStruct((n, in_c), jnp.float32),
        ],
    )


def _mid_body(y1_ref, p_ref, dinv_ref, w1_ref, b1_ref, w2_ref, y2_ref):
    s1 = y1_ref[...] + p_ref[0] + p_ref[1]
    t = dinv_ref[...] * s1
    h = jnp.dot(t, w1_ref[...], precision=_HIGH) + b1_ref[...]
    h = jnp.maximum(h, 0.0)
    y2_ref[...] = dinv_ref[...] * jnp.dot(h, w2_ref[...], precision=_HIGH)


def _make_mid(n, rows, in_c, hid_c, outp, blk):
    grid = -(-rows // blk)
    return pl.pallas_call(
        _mid_body,
        grid=(grid,),
        in_specs=[
            pl.BlockSpec((blk, in_c), lambda i: (i, 0)),
            pl.BlockSpec((NC, blk, in_c), lambda i: (0, i, 0)),
            pl.BlockSpec((blk, 1), lambda i: (i, 0)),
            pl.BlockSpec((in_c, hid_c), lambda i: (0, 0)),
            pl.BlockSpec((1, hid_c), lambda i: (0, 0)),
            pl.BlockSpec((hid_c, outp), lambda i: (0, 0)),
        ],
        out_specs=pl.BlockSpec((blk, outp), lambda i: (i, 0)),
        out_shape=jax.ShapeDtypeStruct((n, outp), jnp.float32),
    )


def _fin_body(out_c, y2_ref, q_ref, dinv_ref, b2_ref, out_ref):
    z = (y2_ref[...] + q_ref[0] + q_ref[1])[:, :out_c]
    z = dinv_ref[...] * z + b2_ref[...]
    m = jnp.max(z, axis=1, keepdims=True)
    e = z - m
    out_ref[...] = e - jnp.log(jnp.sum(jnp.exp(e), axis=1, keepdims=True))


def _make_fin(n, rows, outp, out_c, blk):
    grid = -(-rows // blk)
    return pl.pallas_call(
        partial(_fin_body, out_c),
        grid=(grid,),
        in_specs=[
            pl.BlockSpec((blk, outp), lambda i: (i, 0)),
            pl.BlockSpec((NC, blk, outp), lambda i: (0, i, 0)),
            pl.BlockSpec((blk, 1), lambda i: (i, 0)),
            pl.BlockSpec((1, out_c), lambda i: (0, 0)),
        ],
        out_specs=pl.BlockSpec((blk, out_c), lambda i: (i, 0)),
        out_shape=jax.ShapeDtypeStruct((n, out_c), jnp.float32),
    )


@jax.jit
def kernel(x, edge_index, W1, b1, W2, b2):
    n, in_c = x.shape
    hid_c = W1.shape[1]
    out_c = W2.shape[1]
    outp = -(-out_c // 16) * 16
    e0 = edge_index.shape[1]

    cpw = (-(-e0 // (NW * CHUNK)) + 7) // 8 * 8
    ep = NW * CHUNK * cpw
    rpt = (-(-(n + 1) // NS) + 7) // 8 * 8
    rows = NS * rpt

    src = edge_index[0].astype(jnp.int32)
    dst = edge_index[1].astype(jnp.int32)
    pad = ep - e0
    src2 = jnp.concatenate([src, jnp.zeros((pad,), jnp.int32)]).reshape(-1, CHUNK)
    dst2 = jnp.concatenate([dst, jnp.full((pad,), n, jnp.int32)]).reshape(-1, CHUNK)

    degp = _make_hist(cpw, rows, rpt)(
        dst2, jnp.zeros((rpt, 16), jnp.float32), jnp.ones((CHUNK, 16), jnp.float32))

    blk = 2048
    dinv, y1 = _make_scale(n, rows, in_c, blk)(degp, x)

    p = _make_agg(cpw, rows, rpt, in_c)(
        src2, dst2, y1, jnp.zeros((rpt, in_c), jnp.float32))

    w2p = jnp.concatenate([W2, jnp.zeros((hid_c, outp - out_c), W2.dtype)], axis=1)
    y2 = _make_mid(n, rows, in_c, hid_c, outp, blk)(
        y1, p, dinv, W1, b1.reshape(1, hid_c), w2p)

    q = _make_agg(cpw, rows, rpt, outp)(
        src2, dst2, y2, jnp.zeros((rpt, outp), jnp.float32))

    out = _make_fin(n, rows, outp, out_c, blk)(
        y2, q, dinv, b2.reshape(1, out_c))
    return out


# speedup vs baseline: 12.7498x; 1.1297x over previous
import functools
from functools import partial

import jax
import jax.numpy as jnp
from jax import lax
from jax.experimental import pallas as pl
from jax.experimental.pallas import tpu as pltpu
from jax.experimental.pallas import tpu_sc as plsc

NC = 2
NS = 16
NW = NC * NS
CHUNK = 128

_HIGH = jax.lax.Precision.HIGHEST


def _mesh():
    return plsc.VectorSubcoreMesh(
        core_axis_name="c", subcore_axis_name="s", num_cores=NC, num_subcores=NS
    )


def _hist_body(cpw, rpt, dst_hbm, zeros_hbm, ones_hbm, degp_hbm,
               idx_v, buf_v, acc_sh):
    c = lax.axis_index("c")
    s = lax.axis_index("s")
    wid = s * NC + c
    pltpu.sync_copy(zeros_hbm, acc_sh.at[pl.ds(s * rpt, rpt)])
    pltpu.sync_copy(ones_hbm, buf_v)
    pltpu.sync_copy(dst_hbm.at[pl.ds(wid * cpw, cpw)], idx_v)
    plsc.subcore_barrier()

    def body(j, carry):
        pltpu.sync_copy(buf_v, acc_sh.at[idx_v.at[j]], add=True)
        return carry

    lax.fori_loop(0, cpw, body, 0)
    plsc.subcore_barrier()
    pltpu.sync_copy(acc_sh.at[pl.ds(s * rpt, rpt)],
                    degp_hbm.at[c, pl.ds(s * rpt, rpt)])


def _make_hist(cpw, rows, rpt):
    return pl.kernel(
        partial(_hist_body, cpw, rpt),
        out_type=jax.ShapeDtypeStruct((NC, rows, 16), jnp.float32),
        mesh=_mesh(),
        compiler_params=pltpu.CompilerParams(use_tc_tiling_on_sc=False),
        scratch_types=[
            pltpu.VMEM((cpw, CHUNK), jnp.int32),
            pltpu.VMEM((CHUNK, 16), jnp.float32),
            pltpu.VMEM_SHARED((rows, 16), jnp.float32),
        ],
    )


def _agg_body(cpw, rpt, d, nbuf, ih, src_hbm, dst_hbm, table_hbm, zeros_hbm,
              part_hbm, sidx_v, didx_v, rbuf_v, acc_sh, gsem):
    c = lax.axis_index("c")
    s = lax.axis_index("s")
    wid = s * NC + c
    hpc = cpw // ih
    pltpu.sync_copy(zeros_hbm, acc_sh.at[pl.ds(s * rpt, rpt)])
    plsc.subcore_barrier()

    def gather_start(ch, b):
        pltpu.async_copy(table_hbm.at[sidx_v.at[ch]], rbuf_v.at[b], gsem.at[b])

    def gather_wait(ch, b):
        pltpu.make_async_copy(table_hbm.at[sidx_v.at[ch]], rbuf_v.at[b],
                              gsem.at[b]).wait()

    for h in range(ih):
        base = wid * cpw + h * hpc
        pltpu.sync_copy(src_hbm.at[pl.ds(base, hpc)], sidx_v)
        pltpu.sync_copy(dst_hbm.at[pl.ds(base, hpc)], didx_v)
        for b in range(nbuf):
            gather_start(b, b)

        def rnd(g, carry):
            for b in range(nbuf):
                lch = g * nbuf + b
                gather_wait(lch, b)
                pltpu.sync_copy(rbuf_v.at[b], acc_sh.at[didx_v.at[lch]],
                                add=True)
                gather_start(lch + nbuf, b)
            return carry

        lax.fori_loop(0, hpc // nbuf - 1, rnd, 0)
        for b in range(nbuf):
            lch = hpc - nbuf + b
            gather_wait(lch, b)
            pltpu.sync_copy(rbuf_v.at[b], acc_sh.at[didx_v.at[lch]], add=True)

    plsc.subcore_barrier()
    pltpu.sync_copy(acc_sh.at[pl.ds(s * rpt, rpt)],
                    part_hbm.at[c, pl.ds(s * rpt, rpt)])


def _make_agg(cpw, rows, rpt, d):
    nbuf = 2 if d > 64 else 4
    ih = 2 if d > 64 else 1
    return pl.kernel(
        partial(_agg_body, cpw, rpt, d, nbuf, ih),
        out_type=jax.ShapeDtypeStruct((NC, rows, d), jnp.float32),
        mesh=_mesh(),
        compiler_params=pltpu.CompilerParams(use_tc_tiling_on_sc=False),
        scratch_types=[
            pltpu.VMEM((cpw // ih, CHUNK), jnp.int32),
            pltpu.VMEM((cpw // ih, CHUNK), jnp.int32),
            pltpu.VMEM((nbuf, CHUNK, d), jnp.float32),
            pltpu.VMEM_SHARED((rows, d), jnp.float32),
            pltpu.SemaphoreType.DMA((nbuf,)),
        ],
    )


def _scale_body(degp_ref, x_ref, dinv_ref, y1_ref):
    d = 1.0 + degp_ref[0, :, 0:1] + degp_ref[1, :, 0:1]
    dv = jax.lax.rsqrt(d)
    dinv_ref[...] = dv
    y1_ref[...] = dv * x_ref[...]


def _make_scale(n, rows, in_c, blk):
    grid = -(-rows // blk)
    return pl.pallas_call(
        _scale_body,
        grid=(grid,),
        in_specs=[
            pl.BlockSpec((NC, blk, 16), lambda i: (0, i, 0)),
            pl.BlockSpec((blk, in_c), lambda i: (i, 0)),
        ],
        out_specs=[
            pl.BlockSpec((blk, 1), lambda i: (i, 0)),
            pl.BlockSpec((blk, in_c), lambda i: (i, 0)),
        ],
        out_shape=[
            jax.ShapeDtypeStruct((rows, 1), jnp.float32),
            jax.ShapeDtypeStruct((n, in_c), jnp.float32),
        ],
    )


def _mid_body(y1_ref, p_ref, dinv_ref, w1_ref, b1_ref, w2_ref, y2_ref):
    s1 = y1_ref[...] + p_ref[0] + p_ref[1]
    t = dinv_ref[...] * s1
    h = jnp.dot(t, w1_ref[...], precision=_HIGH) + b1_ref[...]
    h = jnp.maximum(h, 0.0)
    y2_ref[...] = dinv_ref[...] * jnp.dot(h, w2_ref[...], precision=_HIGH)


def _make_mid(n, rows, in_c, hid_c, outp, blk):
    grid = -(-rows // blk)
    return pl.pallas_call(
        _mid_body,
        grid=(grid,),
        in_specs=[
            pl.BlockSpec((blk, in_c), lambda i: (i, 0)),
            pl.BlockSpec((NC, blk, in_c), lambda i: (0, i, 0)),
            pl.BlockSpec((blk, 1), lambda i: (i, 0)),
            pl.BlockSpec((in_c, hid_c), lambda i: (0, 0)),
            pl.BlockSpec((1, hid_c), lambda i: (0, 0)),
            pl.BlockSpec((hid_c, outp), lambda i: (0, 0)),
        ],
        out_specs=pl.BlockSpec((blk, outp), lambda i: (i, 0)),
        out_shape=jax.ShapeDtypeStruct((n, outp), jnp.float32),
    )


def _fin_body(out_c, y2_ref, q_ref, dinv_ref, b2_ref, out_ref):
    z = (y2_ref[...] + q_ref[0] + q_ref[1])[:, :out_c]
    z = dinv_ref[...] * z + b2_ref[...]
    m = jnp.max(z, axis=1, keepdims=True)
    e = z - m
    out_ref[...] = e - jnp.log(jnp.sum(jnp.exp(e), axis=1, keepdims=True))


def _make_fin(n, rows, outp, out_c, blk):
    grid = -(-rows // blk)
    return pl.pallas_call(
        partial(_fin_body, out_c),
        grid=(grid,),
        in_specs=[
            pl.BlockSpec((blk, outp), lambda i: (i, 0)),
            pl.BlockSpec((NC, blk, outp), lambda i: (0, i, 0)),
            pl.BlockSpec((blk, 1), lambda i: (i, 0)),
            pl.BlockSpec((1, out_c), lambda i: (0, 0)),
        ],
        out_specs=pl.BlockSpec((blk, out_c), lambda i: (i, 0)),
        out_shape=jax.ShapeDtypeStruct((n, out_c), jnp.float32),
    )


@jax.jit
def kernel(x, edge_index, W1, b1, W2, b2):
    n, in_c = x.shape
    hid_c = W1.shape[1]
    out_c = W2.shape[1]
    outp = -(-out_c // 16) * 16
    e0 = edge_index.shape[1]

    cpw = (-(-e0 // (NW * CHUNK)) + 7) // 8 * 8
    ep = NW * CHUNK * cpw
    rpt = (-(-(n + 1) // NS) + 7) // 8 * 8
    rows = NS * rpt

    src = edge_index[0].astype(jnp.int32)
    dst = edge_index[1].astype(jnp.int32)
    pad = ep - e0
    pad_dst = n + jnp.arange(pad, dtype=jnp.int32) % (rows - n)
    src2 = jnp.concatenate([src, jnp.zeros((pad,), jnp.int32)]).reshape(-1, CHUNK)
    dst2 = jnp.concatenate([dst, pad_dst]).reshape(-1, CHUNK)

    degp = _make_hist(cpw, rows, rpt)(
        dst2, jnp.zeros((rpt, 16), jnp.float32), jnp.ones((CHUNK, 16), jnp.float32))

    blk = 2048
    dinv, y1 = _make_scale(n, rows, in_c, blk)(degp, x)

    p = _make_agg(cpw, rows, rpt, in_c)(
        src2, dst2, y1, jnp.zeros((rpt, in_c), jnp.float32))

    w2p = jnp.concatenate([W2, jnp.zeros((hid_c, outp - out_c), W2.dtype)], axis=1)
    y2 = _make_mid(n, rows, in_c, hid_c, outp, blk)(
        y1, p, dinv, W1, b1.reshape(1, hid_c), w2p)

    q = _make_agg(cpw, rows, rpt, outp)(
        src2, dst2, y2, jnp.zeros((rpt, outp), jnp.float32))

    out = _make_fin(n, rows, outp, out_c, blk)(
        y2, q, dinv, b2.reshape(1, out_c))
    return out


# speedup vs baseline: 12.7531x; 1.0003x over previous
import functools
from functools import partial

import jax
import jax.numpy as jnp
from jax import lax
from jax.experimental import pallas as pl
from jax.experimental.pallas import tpu as pltpu
from jax.experimental.pallas import tpu_sc as plsc

NC = 2
NS = 16
NW = NC * NS
CHUNK = 128

_HIGH = jax.lax.Precision.HIGHEST


def _mesh():
    return plsc.VectorSubcoreMesh(
        core_axis_name="c", subcore_axis_name="s", num_cores=NC, num_subcores=NS
    )


def _hist_body(cpw, rpt, dst_hbm, zeros_hbm, ones_hbm, degp_hbm,
               idx_v, buf_v, acc_sh):
    c = lax.axis_index("c")
    s = lax.axis_index("s")
    wid = s * NC + c
    pltpu.sync_copy(zeros_hbm, acc_sh.at[pl.ds(s * rpt, rpt)])
    pltpu.sync_copy(ones_hbm, buf_v)
    pltpu.sync_copy(dst_hbm.at[pl.ds(wid * cpw, cpw)], idx_v)
    plsc.subcore_barrier()

    def body(j, carry):
        pltpu.sync_copy(buf_v, acc_sh.at[idx_v.at[j]], add=True)
        return carry

    lax.fori_loop(0, cpw, body, 0)
    plsc.subcore_barrier()
    pltpu.sync_copy(acc_sh.at[pl.ds(s * rpt, rpt)],
                    degp_hbm.at[c, pl.ds(s * rpt, rpt)])


def _make_hist(cpw, rows, rpt):
    return pl.kernel(
        partial(_hist_body, cpw, rpt),
        out_type=jax.ShapeDtypeStruct((NC, rows, 16), jnp.float32),
        mesh=_mesh(),
        compiler_params=pltpu.CompilerParams(use_tc_tiling_on_sc=False),
        scratch_types=[
            pltpu.VMEM((cpw, CHUNK), jnp.int32),
            pltpu.VMEM((CHUNK, 16), jnp.float32),
            pltpu.VMEM_SHARED((rows, 16), jnp.float32),
        ],
    )


def _agg_body(cpw, rpt, d, nbuf, ih, src_hbm, dst_hbm, table_hbm, zeros_hbm,
              part_hbm, sidx_v, didx_v, rbuf_v, acc_sh, gsem):
    c = lax.axis_index("c")
    s = lax.axis_index("s")
    wid = s * NC + c
    hpc = cpw // ih
    pltpu.sync_copy(zeros_hbm, acc_sh.at[pl.ds(s * rpt, rpt)])
    plsc.subcore_barrier()

    def gather_start(ch, b):
        pltpu.async_copy(table_hbm.at[sidx_v.at[ch]], rbuf_v.at[b], gsem.at[b])

    def gather_wait(ch, b):
        pltpu.make_async_copy(table_hbm.at[sidx_v.at[ch]], rbuf_v.at[b],
                              gsem.at[b]).wait()

    for h in range(ih):
        base = wid * cpw + h * hpc
        pltpu.sync_copy(src_hbm.at[pl.ds(base, hpc)], sidx_v)
        pltpu.sync_copy(dst_hbm.at[pl.ds(base, hpc)], didx_v)
        for b in range(nbuf):
            gather_start(b, b)

        def rnd(g, carry):
            for b in range(nbuf):
                lch = g * nbuf + b
                gather_wait(lch, b)
                pltpu.sync_copy(rbuf_v.at[b], acc_sh.at[didx_v.at[lch]],
                                add=True)
                gather_start(lch + nbuf, b)
            return carry

        lax.fori_loop(0, hpc // nbuf - 1, rnd, 0)
        for b in range(nbuf):
            lch = hpc - nbuf + b
            gather_wait(lch, b)
            pltpu.sync_copy(rbuf_v.at[b], acc_sh.at[didx_v.at[lch]], add=True)

    plsc.subcore_barrier()
    pltpu.sync_copy(acc_sh.at[pl.ds(s * rpt, rpt)],
                    part_hbm.at[c, pl.ds(s * rpt, rpt)])


def _make_agg(cpw, rows, rpt, d):
    nbuf = 2 if d > 64 else 8
    ih = 2 if d > 64 else 1
    return pl.kernel(
        partial(_agg_body, cpw, rpt, d, nbuf, ih),
        out_type=jax.ShapeDtypeStruct((NC, rows, d), jnp.float32),
        mesh=_mesh(),
        compiler_params=pltpu.CompilerParams(use_tc_tiling_on_sc=False),
        scratch_types=[
            pltpu.VMEM((cpw // ih, CHUNK), jnp.int32),
            pltpu.VMEM((cpw // ih, CHUNK), jnp.int32),
            pltpu.VMEM((nbuf, CHUNK, d), jnp.float32),
            pltpu.VMEM_SHARED((rows, d), jnp.float32),
            pltpu.SemaphoreType.DMA((nbuf,)),
        ],
    )


def _scale_body(degp_ref, x_ref, dinv_ref, y1_ref):
    d = 1.0 + degp_ref[0, :, 0:1] + degp_ref[1, :, 0:1]
    dv = jax.lax.rsqrt(d)
    dinv_ref[...] = dv
    y1_ref[...] = dv * x_ref[...]


def _make_scale(n, rows, in_c, blk):
    grid = -(-rows // blk)
    return pl.pallas_call(
        _scale_body,
        grid=(grid,),
        in_specs=[
            pl.BlockSpec((NC, blk, 16), lambda i: (0, i, 0)),
            pl.BlockSpec((blk, in_c), lambda i: (i, 0)),
        ],
        out_specs=[
            pl.BlockSpec((blk, 1), lambda i: (i, 0)),
            pl.BlockSpec((blk, in_c), lambda i: (i, 0)),
        ],
        out_shape=[
            jax.ShapeDtypeStruct((rows, 1), jnp.float32),
            jax.ShapeDtypeStruct((n, in_c), jnp.float32),
        ],
    )


def _mid_body(y1_ref, p_ref, dinv_ref, w1_ref, b1_ref, w2_ref, y2_ref):
    s1 = y1_ref[...] + p_ref[0] + p_ref[1]
    t = dinv_ref[...] * s1
    h = jnp.dot(t, w1_ref[...], precision=_HIGH) + b1_ref[...]
    h = jnp.maximum(h, 0.0)
    y2_ref[...] = dinv_ref[...] * jnp.dot(h, w2_ref[...], precision=_HIGH)


def _make_mid(n, rows, in_c, hid_c, outp, blk):
    grid = -(-rows // blk)
    return pl.pallas_call(
        _mid_body,
        grid=(grid,),
        in_specs=[
            pl.BlockSpec((blk, in_c), lambda i: (i, 0)),
            pl.BlockSpec((NC, blk, in_c), lambda i: (0, i, 0)),
            pl.BlockSpec((blk, 1), lambda i: (i, 0)),
            pl.BlockSpec((in_c, hid_c), lambda i: (0, 0)),
            pl.BlockSpec((1, hid_c), lambda i: (0, 0)),
            pl.BlockSpec((hid_c, outp), lambda i: (0, 0)),
        ],
        out_specs=pl.BlockSpec((blk, outp), lambda i: (i, 0)),
        out_shape=jax.ShapeDtypeStruct((n, outp), jnp.float32),
    )


def _fin_body(out_c, y2_ref, q_ref, dinv_ref, b2_ref, out_ref):
    z = (y2_ref[...] + q_ref[0] + q_ref[1])[:, :out_c]
    z = dinv_ref[...] * z + b2_ref[...]
    m = jnp.max(z, axis=1, keepdims=True)
    e = z - m
    out_ref[...] = e - jnp.log(jnp.sum(jnp.exp(e), axis=1, keepdims=True))


def _make_fin(n, rows, outp, out_c, blk):
    grid = -(-rows // blk)
    return pl.pallas_call(
        partial(_fin_body, out_c),
        grid=(grid,),
        in_specs=[
            pl.BlockSpec((blk, outp), lambda i: (i, 0)),
            pl.BlockSpec((NC, blk, outp), lambda i: (0, i, 0)),
            pl.BlockSpec((blk, 1), lambda i: (i, 0)),
            pl.BlockSpec((1, out_c), lambda i: (0, 0)),
        ],
        out_specs=pl.BlockSpec((blk, out_c), lambda i: (i, 0)),
        out_shape=jax.ShapeDtypeStruct((n, out_c), jnp.float32),
    )


@jax.jit
def kernel(x, edge_index, W1, b1, W2, b2):
    n, in_c = x.shape
    hid_c = W1.shape[1]
    out_c = W2.shape[1]
    outp = -(-out_c // 16) * 16
    e0 = edge_index.shape[1]

    cpw = (-(-e0 // (NW * CHUNK)) + 7) // 8 * 8
    ep = NW * CHUNK * cpw
    rpt = (-(-(n + 1) // NS) + 7) // 8 * 8
    rows = NS * rpt

    src = edge_index[0].astype(jnp.int32)
    dst = edge_index[1].astype(jnp.int32)
    pad = ep - e0
    pad_dst = n + jnp.arange(pad, dtype=jnp.int32) % (rows - n)
    src2 = jnp.concatenate([src, jnp.zeros((pad,), jnp.int32)]).reshape(-1, CHUNK)
    dst2 = jnp.concatenate([dst, pad_dst]).reshape(-1, CHUNK)

    degp = _make_hist(cpw, rows, rpt)(
        dst2, jnp.zeros((rpt, 16), jnp.float32), jnp.ones((CHUNK, 16), jnp.float32))

    blk = 2048
    dinv, y1 = _make_scale(n, rows, in_c, blk)(degp, x)

    p = _make_agg(cpw, rows, rpt, in_c)(
        src2, dst2, y1, jnp.zeros((rpt, in_c), jnp.float32))

    w2p = jnp.concatenate([W2, jnp.zeros((hid_c, outp - out_c), W2.dtype)], axis=1)
    y2 = _make_mid(n, rows, in_c, hid_c, outp, blk)(
        y1, p, dinv, W1, b1.reshape(1, hid_c), w2p)

    q = _make_agg(cpw, rows, rpt, outp)(
        src2, dst2, y2, jnp.zeros((rpt, outp), jnp.float32))

    out = _make_fin(n, rows, outp, out_c, blk)(
        y2, q, dinv, b2.reshape(1, out_c))
    return out


# speedup vs baseline: 36.5328x; 2.8646x over previous
import functools
from functools import partial

import jax
import jax.numpy as jnp
from jax import lax
from jax.experimental import pallas as pl
from jax.experimental.pallas import tpu as pltpu
from jax.experimental.pallas import tpu_sc as plsc

NC = 2
NS = 16
NW = NC * NS
CHUNK = 128

_HIGH = jax.lax.Precision.HIGHEST


def _mesh():
    return plsc.VectorSubcoreMesh(
        core_axis_name="c", subcore_axis_name="s", num_cores=NC, num_subcores=NS
    )


def _hist_body(cpw, rpt, dst_hbm, zeros_hbm, ones_hbm, degp_hbm,
               idx_v, buf_v, acc_sh):
    c = lax.axis_index("c")
    s = lax.axis_index("s")
    wid = s * NC + c
    pltpu.sync_copy(zeros_hbm, acc_sh.at[pl.ds(s * rpt, rpt)])
    pltpu.sync_copy(ones_hbm, buf_v)
    pltpu.sync_copy(dst_hbm.at[pl.ds(wid * cpw, cpw)], idx_v)
    plsc.subcore_barrier()

    def body(j, carry):
        pltpu.sync_copy(buf_v, acc_sh.at[idx_v.at[j]], add=True)
        return carry

    lax.fori_loop(0, cpw, body, 0)
    plsc.subcore_barrier()
    pltpu.sync_copy(acc_sh.at[pl.ds(s * rpt, rpt)],
                    degp_hbm.at[c, pl.ds(s * rpt, rpt)])


def _make_hist(cpw, rows, rpt):
    return pl.kernel(
        partial(_hist_body, cpw, rpt),
        out_type=jax.ShapeDtypeStruct((NC, rows, 16), jnp.float32),
        mesh=_mesh(),
        compiler_params=pltpu.CompilerParams(use_tc_tiling_on_sc=False),
        scratch_types=[
            pltpu.VMEM((cpw, CHUNK), jnp.int32),
            pltpu.VMEM((CHUNK, 16), jnp.float32),
            pltpu.VMEM_SHARED((rows, 16), jnp.float32),
        ],
    )


def _agg_body(cpw, rpt, d, nbuf, ih, src_hbm, dst_hbm, table_hbm, zeros_hbm,
              part_hbm, sidx_v, didx_v, rbuf_v, acc_sh, gsem):
    c = lax.axis_index("c")
    s = lax.axis_index("s")
    wid = s * NC + c
    hpc = cpw // ih
    pltpu.sync_copy(zeros_hbm, acc_sh.at[pl.ds(s * rpt, rpt)])
    plsc.subcore_barrier()

    def gather_start(ch, b):
        pltpu.async_copy(table_hbm.at[sidx_v.at[ch]], rbuf_v.at[b], gsem.at[b])

    def gather_wait(ch, b):
        pltpu.make_async_copy(table_hbm.at[sidx_v.at[ch]], rbuf_v.at[b],
                              gsem.at[b]).wait()

    for h in range(ih):
        base = wid * cpw + h * hpc
        pltpu.sync_copy(src_hbm.at[pl.ds(base, hpc)], sidx_v)
        pltpu.sync_copy(dst_hbm.at[pl.ds(base, hpc)], didx_v)
        for b in range(nbuf):
            gather_start(b, b)

        def rnd(g, carry):
            for b in range(nbuf):
                lch = g * nbuf + b
                gather_wait(lch, b)
                pltpu.sync_copy(rbuf_v.at[b], acc_sh.at[didx_v.at[lch]],
                                add=True)
                gather_start(lch + nbuf, b)
            return carry

        lax.fori_loop(0, hpc // nbuf - 1, rnd, 0)
        for b in range(nbuf):
            lch = hpc - nbuf + b
            gather_wait(lch, b)
            pltpu.sync_copy(rbuf_v.at[b], acc_sh.at[didx_v.at[lch]], add=True)

    plsc.subcore_barrier()
    pltpu.sync_copy(acc_sh.at[pl.ds(s * rpt, rpt)],
                    part_hbm.at[c, pl.ds(s * rpt, rpt)])


def _make_agg(cpw, rows, rpt, d):
    nbuf = 2 if d > 64 else 8
    ih = 2 if d > 64 else 1
    return pl.kernel(
        partial(_agg_body, cpw, rpt, d, nbuf, ih),
        out_type=jax.ShapeDtypeStruct((NC, rows, d), jnp.float32),
        mesh=_mesh(),
        compiler_params=pltpu.CompilerParams(use_tc_tiling_on_sc=False),
        scratch_types=[
            pltpu.VMEM((cpw // ih, CHUNK), jnp.int32),
            pltpu.VMEM((cpw // ih, CHUNK), jnp.int32),
            pltpu.VMEM((nbuf, CHUNK, d), jnp.float32),
            pltpu.VMEM_SHARED((rows, d), jnp.float32),
            pltpu.SemaphoreType.DMA((nbuf,)),
        ],
    )


def _scale_body(degp_ref, x_ref, dinv_ref, y1_ref):
    d = 1.0 + degp_ref[0, :, 0:1] + degp_ref[1, :, 0:1]
    dv = jax.lax.rsqrt(d)
    dinv_ref[...] = dv
    y1_ref[...] = dv * x_ref[...]


def _make_scale(n, rows, in_c, blk):
    grid = -(-rows // blk)
    return pl.pallas_call(
        _scale_body,
        grid=(grid,),
        in_specs=[
            pl.BlockSpec((NC, blk, 16), lambda i: (0, i, 0)),
            pl.BlockSpec((blk, in_c), lambda i: (i, 0)),
        ],
        out_specs=[
            pl.BlockSpec((blk, 1), lambda i: (i, 0)),
            pl.BlockSpec((blk, in_c), lambda i: (i, 0)),
        ],
        out_shape=[
            jax.ShapeDtypeStruct((rows, 1), jnp.float32),
            jax.ShapeDtypeStruct((n, in_c), jnp.float32),
        ],
    )


def _mid_body(y1_ref, p_ref, dinv_ref, w1_ref, b1_ref, w2_ref, y2_ref):
    s1 = y1_ref[...] + p_ref[0] + p_ref[1]
    t = dinv_ref[...] * s1
    h = jnp.dot(t, w1_ref[...], precision=_HIGH) + b1_ref[...]
    h = jnp.maximum(h, 0.0)
    y2_ref[...] = dinv_ref[...] * jnp.dot(h, w2_ref[...], precision=_HIGH)


def _make_mid(n, rows, in_c, hid_c, outp, blk):
    grid = -(-rows // blk)
    return pl.pallas_call(
        _mid_body,
        grid=(grid,),
        in_specs=[
            pl.BlockSpec((blk, in_c), lambda i: (i, 0)),
            pl.BlockSpec((NC, blk, in_c), lambda i: (0, i, 0)),
            pl.BlockSpec((blk, 1), lambda i: (i, 0)),
            pl.BlockSpec((in_c, hid_c), lambda i: (0, 0)),
            pl.BlockSpec((1, hid_c), lambda i: (0, 0)),
            pl.BlockSpec((hid_c, outp), lambda i: (0, 0)),
        ],
        out_specs=pl.BlockSpec((blk, outp), lambda i: (i, 0)),
        out_shape=jax.ShapeDtypeStruct((n, outp), jnp.float32),
    )


def _fin_body(out_c, y2_ref, q_ref, dinv_ref, b2_ref, out_ref):
    z = (y2_ref[...] + q_ref[0] + q_ref[1])[:, :out_c]
    z = dinv_ref[...] * z + b2_ref[...]
    m = jnp.max(z, axis=1, keepdims=True)
    e = z - m
    out_ref[...] = e - jnp.log(jnp.sum(jnp.exp(e), axis=1, keepdims=True))


def _make_fin(n, rows, outp, out_c, blk):
    grid = -(-rows // blk)
    return pl.pallas_call(
        partial(_fin_body, out_c),
        grid=(grid,),
        in_specs=[
            pl.BlockSpec((blk, outp), lambda i: (i, 0)),
            pl.BlockSpec((NC, blk, outp), lambda i: (0, i, 0)),
            pl.BlockSpec((blk, 1), lambda i: (i, 0)),
            pl.BlockSpec((1, out_c), lambda i: (0, 0)),
        ],
        out_specs=pl.BlockSpec((blk, out_c), lambda i: (i, 0)),
        out_shape=jax.ShapeDtypeStruct((n, out_c), jnp.float32),
    )


@jax.jit
def kernel(x, edge_index, W1, b1, W2, b2):
    n, in_c = x.shape
    hid_c = W1.shape[1]
    out_c = W2.shape[1]
    outp = -(-out_c // 16) * 16
    e0 = edge_index.shape[1]

    cpw = (-(-e0 // (NW * CHUNK)) + 7) // 8 * 8
    ep = NW * CHUNK * cpw
    rpt = (-(-(n + 1) // NS) + 7) // 8 * 8
    rows = NS * rpt

    src = edge_index[0].astype(jnp.int32)
    dst = edge_index[1].astype(jnp.int32)
    pad = ep - e0
    pad_dst = n + jnp.arange(pad, dtype=jnp.int32) % (rows - n)
    pad_src = jnp.arange(pad, dtype=jnp.int32) % n
    src2 = jnp.concatenate([src, pad_src]).reshape(-1, CHUNK)
    dst2 = jnp.concatenate([dst, pad_dst]).reshape(-1, CHUNK)

    degp = _make_hist(cpw, rows, rpt)(
        dst2, jnp.zeros((rpt, 16), jnp.float32), jnp.ones((CHUNK, 16), jnp.float32))

    blk = 2048
    dinv, y1 = _make_scale(n, rows, in_c, blk)(degp, x)

    p = _make_agg(cpw, rows, rpt, in_c)(
        src2, dst2, y1, jnp.zeros((rpt, in_c), jnp.float32))

    w2p = jnp.concatenate([W2, jnp.zeros((hid_c, outp - out_c), W2.dtype)], axis=1)
    y2 = _make_mid(n, rows, in_c, hid_c, outp, blk)(
        y1, p, dinv, W1, b1.reshape(1, hid_c), w2p)

    q = _make_agg(cpw, rows, rpt, outp)(
        src2, dst2, y2, jnp.zeros((rpt, outp), jnp.float32))

    out = _make_fin(n, rows, outp, out_c, blk)(
        y2, q, dinv, b2.reshape(1, out_c))
    return out


# speedup vs baseline: 41.9535x; 1.1484x over previous
import functools
from functools import partial

import jax
import jax.numpy as jnp
from jax import lax
from jax.experimental import pallas as pl
from jax.experimental.pallas import tpu as pltpu
from jax.experimental.pallas import tpu_sc as plsc

NC = 2
NS = 16
NW = NC * NS
CHUNK = 128

_HIGH = jax.lax.Precision.HIGHEST


def _mesh():
    return plsc.VectorSubcoreMesh(
        core_axis_name="c", subcore_axis_name="s", num_cores=NC, num_subcores=NS
    )


def _hist_body(cpw, rpt, dst_hbm, zeros_hbm, ones_hbm, degp_hbm,
               idx_v, buf_v, acc_sh):
    c = lax.axis_index("c")
    s = lax.axis_index("s")
    wid = s * NC + c
    pltpu.sync_copy(zeros_hbm, acc_sh.at[pl.ds(s * rpt, rpt)])
    pltpu.sync_copy(ones_hbm, buf_v)
    pltpu.sync_copy(dst_hbm.at[pl.ds(wid * cpw, cpw)], idx_v)
    plsc.subcore_barrier()

    def body(j, carry):
        pltpu.sync_copy(buf_v, acc_sh.at[idx_v.at[j]], add=True)
        return carry

    lax.fori_loop(0, cpw, body, 0)
    plsc.subcore_barrier()
    pltpu.sync_copy(acc_sh.at[pl.ds(s * rpt, rpt)],
                    degp_hbm.at[c, pl.ds(s * rpt, rpt)])


def _make_hist(cpw, rows, rpt):
    return pl.kernel(
        partial(_hist_body, cpw, rpt),
        out_type=jax.ShapeDtypeStruct((NC, rows, 16), jnp.float32),
        mesh=_mesh(),
        compiler_params=pltpu.CompilerParams(use_tc_tiling_on_sc=False),
        scratch_types=[
            pltpu.VMEM((cpw, CHUNK), jnp.int32),
            pltpu.VMEM((CHUNK, 16), jnp.float32),
            pltpu.VMEM_SHARED((rows, 16), jnp.float32),
        ],
    )


def _agg_body(cpw, rpt, d, nbuf, ih, src_hbm, dst_hbm, table_hbm, zeros_hbm,
              part_hbm, sidx_v, didx_v, rbuf_v, acc_sh, gsem):
    c = lax.axis_index("c")
    s = lax.axis_index("s")
    wid = s * NC + c
    hpc = cpw // ih
    pltpu.sync_copy(zeros_hbm, acc_sh.at[pl.ds(s * rpt, rpt)])
    plsc.subcore_barrier()

    def gather_start(ch, b):
        pltpu.async_copy(table_hbm.at[sidx_v.at[ch]], rbuf_v.at[b], gsem.at[b])

    def gather_wait(ch, b):
        pltpu.make_async_copy(table_hbm.at[sidx_v.at[ch]], rbuf_v.at[b],
                              gsem.at[b]).wait()

    for h in range(ih):
        base = wid * cpw + h * hpc
        pltpu.sync_copy(src_hbm.at[pl.ds(base, hpc)], sidx_v)
        pltpu.sync_copy(dst_hbm.at[pl.ds(base, hpc)], didx_v)
        for b in range(nbuf):
            gather_start(b, b)

        def rnd(g, carry):
            for b in range(nbuf):
                lch = g * nbuf + b
                gather_wait(lch, b)
                pltpu.sync_copy(rbuf_v.at[b], acc_sh.at[didx_v.at[lch]],
                                add=True)
                gather_start(lch + nbuf, b)
            return carry

        lax.fori_loop(0, hpc // nbuf - 1, rnd, 0)
        for b in range(nbuf):
            lch = hpc - nbuf + b
            gather_wait(lch, b)
            pltpu.sync_copy(rbuf_v.at[b], acc_sh.at[didx_v.at[lch]], add=True)

    plsc.subcore_barrier()
    pltpu.sync_copy(acc_sh.at[pl.ds(s * rpt, rpt)],
                    part_hbm.at[c, pl.ds(s * rpt, rpt)])


def _make_agg(cpw, rows, rpt, d):
    nbuf = 4 if d > 64 else 8
    ih = 2 if d > 64 else 1
    return pl.kernel(
        partial(_agg_body, cpw, rpt, d, nbuf, ih),
        out_type=jax.ShapeDtypeStruct((NC, rows, d), jnp.bfloat16),
        mesh=_mesh(),
        compiler_params=pltpu.CompilerParams(use_tc_tiling_on_sc=False),
        scratch_types=[
            pltpu.VMEM((cpw // ih, CHUNK), jnp.int32),
            pltpu.VMEM((cpw // ih, CHUNK), jnp.int32),
            pltpu.VMEM((nbuf, CHUNK, d), jnp.bfloat16),
            pltpu.VMEM_SHARED((rows, d), jnp.bfloat16),
            pltpu.SemaphoreType.DMA((nbuf,)),
        ],
    )


def _scale_body(degp_ref, x_ref, dinv_ref, y1_ref):
    d = 1.0 + degp_ref[0, :, 0:1] + degp_ref[1, :, 0:1]
    dv = jax.lax.rsqrt(d)
    dinv_ref[...] = dv
    y1_ref[...] = (dv * x_ref[...]).astype(jnp.bfloat16)


def _make_scale(n, rows, in_c, blk):
    grid = -(-rows // blk)
    return pl.pallas_call(
        _scale_body,
        grid=(grid,),
        in_specs=[
            pl.BlockSpec((NC, blk, 16), lambda i: (0, i, 0)),
            pl.BlockSpec((blk, in_c), lambda i: (i, 0)),
        ],
        out_specs=[
            pl.BlockSpec((blk, 1), lambda i: (i, 0)),
            pl.BlockSpec((blk, in_c), lambda i: (i, 0)),
        ],
        out_shape=[
            jax.ShapeDtypeStruct((rows, 1), jnp.float32),
            jax.ShapeDtypeStruct((n, in_c), jnp.bfloat16),
        ],
    )


def _mid_body(y1_ref, p_ref, dinv_ref, w1_ref, b1_ref, w2_ref, y2_ref):
    s1 = (y1_ref[...].astype(jnp.float32)
          + p_ref[0].astype(jnp.float32) + p_ref[1].astype(jnp.float32))
    t = dinv_ref[...] * s1
    h = jnp.dot(t, w1_ref[...], precision=_HIGH) + b1_ref[...]
    h = jnp.maximum(h, 0.0)
    y2 = dinv_ref[...] * jnp.dot(h, w2_ref[...], precision=_HIGH)
    y2_ref[...] = y2.astype(jnp.bfloat16)


def _make_mid(n, rows, in_c, hid_c, outp, blk):
    grid = -(-rows // blk)
    return pl.pallas_call(
        _mid_body,
        grid=(grid,),
        in_specs=[
            pl.BlockSpec((blk, in_c), lambda i: (i, 0)),
            pl.BlockSpec((NC, blk, in_c), lambda i: (0, i, 0)),
            pl.BlockSpec((blk, 1), lambda i: (i, 0)),
            pl.BlockSpec((in_c, hid_c), lambda i: (0, 0)),
            pl.BlockSpec((1, hid_c), lambda i: (0, 0)),
            pl.BlockSpec((hid_c, outp), lambda i: (0, 0)),
        ],
        out_specs=pl.BlockSpec((blk, outp), lambda i: (i, 0)),
        out_shape=jax.ShapeDtypeStruct((n, outp), jnp.bfloat16),
    )


def _fin_body(out_c, y2_ref, q_ref, dinv_ref, b2_ref, out_ref):
    z = (y2_ref[...].astype(jnp.float32) + q_ref[0].astype(jnp.float32)
         + q_ref[1].astype(jnp.float32))[:, :out_c]
    z = dinv_ref[...] * z + b2_ref[...]
    m = jnp.max(z, axis=1, keepdims=True)
    e = z - m
    out_ref[...] = e - jnp.log(jnp.sum(jnp.exp(e), axis=1, keepdims=True))


def _make_fin(n, rows, outp, out_c, blk):
    grid = -(-rows // blk)
    return pl.pallas_call(
        partial(_fin_body, out_c),
        grid=(grid,),
        in_specs=[
            pl.BlockSpec((blk, outp), lambda i: (i, 0)),
            pl.BlockSpec((NC, blk, outp), lambda i: (0, i, 0)),
            pl.BlockSpec((blk, 1), lambda i: (i, 0)),
            pl.BlockSpec((1, out_c), lambda i: (0, 0)),
        ],
        out_specs=pl.BlockSpec((blk, out_c), lambda i: (i, 0)),
        out_shape=jax.ShapeDtypeStruct((n, out_c), jnp.float32),
    )


@jax.jit
def kernel(x, edge_index, W1, b1, W2, b2):
    n, in_c = x.shape
    hid_c = W1.shape[1]
    out_c = W2.shape[1]
    outp = -(-out_c // 32) * 32
    e0 = edge_index.shape[1]

    cpw = (-(-e0 // (NW * CHUNK)) + 7) // 8 * 8
    ep = NW * CHUNK * cpw
    rpt = (-(-(n + 1) // NS) + 7) // 8 * 8
    rows = NS * rpt

    src = edge_index[0].astype(jnp.int32)
    dst = edge_index[1].astype(jnp.int32)
    pad = ep - e0
    pad_dst = n + jnp.arange(pad, dtype=jnp.int32) % (rows - n)
    pad_src = jnp.arange(pad, dtype=jnp.int32) % n
    src2 = jnp.concatenate([src, pad_src]).reshape(-1, CHUNK)
    dst2 = jnp.concatenate([dst, pad_dst]).reshape(-1, CHUNK)

    degp = _make_hist(cpw, rows, rpt)(
        dst2, jnp.zeros((rpt, 16), jnp.float32), jnp.ones((CHUNK, 16), jnp.float32))

    blk = 2048
    dinv, y1 = _make_scale(n, rows, in_c, blk)(degp, x)

    p = _make_agg(cpw, rows, rpt, in_c)(
        src2, dst2, y1, jnp.zeros((rpt, in_c), jnp.bfloat16))

    w2p = jnp.concatenate([W2, jnp.zeros((hid_c, outp - out_c), W2.dtype)], axis=1)
    y2 = _make_mid(n, rows, in_c, hid_c, outp, blk)(
        y1, p, dinv, W1, b1.reshape(1, hid_c), w2p)

    q = _make_agg(cpw, rows, rpt, outp)(
        src2, dst2, y2, jnp.zeros((rpt, outp), jnp.bfloat16))

    out = _make_fin(n, rows, outp, out_c, blk)(
        y2, q, dinv, b2.reshape(1, out_c))
    return out
